# Initial kernel scaffold; baseline (speedup 1.0000x reference)
import jax
import jax.numpy as jnp
from jax.experimental import pallas as pl


def kernel(x, edge_index, W1, b1, W2, b2):
    raise NotImplementedError("write your pallas kernel here")


# speedup vs baseline: 21.3984x; 21.3984x over previous
import jax
import jax.numpy as jnp
from jax import lax
from jax.experimental import pallas as pl
from jax.experimental.pallas import tpu as pltpu
from jax.experimental.pallas import tpu_sc as plsc

N = 10000
E = 320000
D = 128
DH = D // 2
NC = 2
NS = 16
NW = NC * NS
EPW = E // NW
K = 80
NCHUNK = EPW // K
SB = 624
LAST_BASE = (NS - 1) * SB
LAST_SIZE = N - LAST_BASE
DEGW = 16

_MESH = plsc.VectorSubcoreMesh(core_axis_name="c", subcore_axis_name="s")


def _deg_body(col_hbm, deg_hbm, col_v, ones_v, zbuf_v, acc_sh, sem):
    c = lax.axis_index("c")
    s = lax.axis_index("s")
    wid = s * NC + c

    def fill_ones(i, carry):
        ones_v[i, :] = jnp.full((DEGW,), 1.0, jnp.float32)
        return carry

    lax.fori_loop(0, K, fill_ones, 0)

    def fill_zero(i, carry):
        zbuf_v[i, :] = jnp.zeros((DEGW,), jnp.float32)
        return carry

    lax.fori_loop(0, LAST_SIZE, fill_zero, 0)

    base = pl.multiple_of(s * SB, 8)

    @pl.when(s < NS - 1)
    def _():
        pltpu.sync_copy(zbuf_v.at[pl.ds(0, SB)], acc_sh.at[pl.ds(base, SB)])

    @pl.when(s == NS - 1)
    def _():
        pltpu.sync_copy(zbuf_v, acc_sh.at[pl.ds(LAST_BASE, LAST_SIZE)])

    pltpu.sync_copy(col_hbm.at[wid], col_v)
    plsc.subcore_barrier()

    def chunk(j, carry):
        pltpu.sync_copy(ones_v, acc_sh.at[col_v.at[j]], add=True)
        return carry

    lax.fori_loop(0, NCHUNK, chunk, 0)
    plsc.subcore_barrier()

    @pl.when(s < NS - 1)
    def _():
        pltpu.sync_copy(acc_sh.at[pl.ds(base, SB)],
                        deg_hbm.at[c, pl.ds(base, SB)])

    @pl.when(s == NS - 1)
    def _():
        pltpu.sync_copy(acc_sh.at[pl.ds(LAST_BASE, LAST_SIZE)],
                        deg_hbm.at[c, pl.ds(LAST_BASE, LAST_SIZE)])


_SC_PARAMS = pltpu.CompilerParams(use_tc_tiling_on_sc=False)

_deg_call = pl.kernel(
    _deg_body,
    out_type=jax.ShapeDtypeStruct((NC, N, DEGW), jnp.float32),
    mesh=_MESH,
    compiler_params=_SC_PARAMS,
    scratch_types=[
        pltpu.VMEM((NCHUNK, K), jnp.int32),
        pltpu.VMEM((K, DEGW), jnp.float32),
        pltpu.VMEM((LAST_SIZE, DEGW), jnp.float32),
        pltpu.VMEM_SHARED((N, DEGW), jnp.float32),
        pltpu.SemaphoreType.DMA,
    ],
)


def _agg_body(hs_hbm, row_hbm, col_hbm, acc_hbm,
              row_v, col_v, buf_v, acc_sh, sem0, sem1):
    c = lax.axis_index("c")
    s = lax.axis_index("s")
    wid = s * NC + c

    base = pl.multiple_of(s * SB, 8)

    @pl.when(s < NS - 1)
    def _():
        pltpu.sync_copy(hs_hbm.at[pl.ds(base, SB)],
                        acc_sh.at[pl.ds(base, SB)])

    @pl.when(s == NS - 1)
    def _():
        pltpu.sync_copy(hs_hbm.at[pl.ds(LAST_BASE, LAST_SIZE)],
                        acc_sh.at[pl.ds(LAST_BASE, LAST_SIZE)])

    pltpu.sync_copy(row_hbm.at[wid], row_v)
    pltpu.sync_copy(col_hbm.at[wid], col_v)
    plsc.subcore_barrier()

    pltpu.async_copy(hs_hbm.at[row_v.at[0]], buf_v.at[0], sem0)
    pltpu.async_copy(hs_hbm.at[row_v.at[1]], buf_v.at[1], sem1)

    def pair(p, carry):
        j0 = 2 * p
        pltpu.make_async_copy(hs_hbm.at[row_v.at[j0]], buf_v.at[0], sem0).wait()
        pltpu.sync_copy(buf_v.at[0], acc_sh.at[col_v.at[j0]], add=True)

        @pl.when(j0 + 2 < NCHUNK)
        def _():
            pltpu.async_copy(hs_hbm.at[row_v.at[j0 + 2]], buf_v.at[0], sem0)

        j1 = j0 + 1
        pltpu.make_async_copy(hs_hbm.at[row_v.at[j1]], buf_v.at[1], sem1).wait()
        pltpu.sync_copy(buf_v.at[1], acc_sh.at[col_v.at[j1]], add=True)

        @pl.when(j1 + 2 < NCHUNK)
        def _():
            pltpu.async_copy(hs_hbm.at[row_v.at[j1 + 2]], buf_v.at[1], sem1)

        return carry

    lax.fori_loop(0, NCHUNK // 2, pair, 0)
    if NCHUNK % 2 == 1:
        j = NCHUNK - 1
        pltpu.make_async_copy(hs_hbm.at[row_v.at[j]], buf_v.at[0], sem0).wait()
        pltpu.sync_copy(buf_v.at[0], acc_sh.at[col_v.at[j]], add=True)

    plsc.subcore_barrier()

    @pl.when(s < NS - 1)
    def _():
        pltpu.sync_copy(acc_sh.at[pl.ds(base, SB)],
                        acc_hbm.at[c, pl.ds(base, SB)])

    @pl.when(s == NS - 1)
    def _():
        pltpu.sync_copy(acc_sh.at[pl.ds(LAST_BASE, LAST_SIZE)],
                        acc_hbm.at[c, pl.ds(LAST_BASE, LAST_SIZE)])


_agg_call = pl.kernel(
    _agg_body,
    out_type=jax.ShapeDtypeStruct((NC, N, DH), jnp.float32),
    mesh=_MESH,
    compiler_params=_SC_PARAMS,
    scratch_types=[
        pltpu.VMEM((NCHUNK, K), jnp.int32),
        pltpu.VMEM((NCHUNK, K), jnp.int32),
        pltpu.VMEM((2, K, DH), jnp.float32),
        pltpu.VMEM_SHARED((N, DH), jnp.float32),
        pltpu.SemaphoreType.DMA,
        pltpu.SemaphoreType.DMA,
    ],
)


BR = 1000


def _dis_from(dp_ref):
    deg = dp_ref[0, :, 0] + dp_ref[1, :, 0] + 1.0
    return lax.rsqrt(deg)


def _lin1_body(dp_ref, x_ref, w_ref, oa_ref, ob_ref):
    dis = _dis_from(dp_ref)
    res = jnp.dot(x_ref[...], w_ref[...],
                  preferred_element_type=jnp.float32) * dis[:, None]
    oa_ref[...] = res[:, :DH]
    ob_ref[...] = res[:, DH:]


def _lin2_body(dp_ref, aa_ref, ab_ref, ha_ref, hb_ref, b_ref, w_ref,
               oa_ref, ob_ref):
    dis = _dis_from(dp_ref)
    agg = jnp.concatenate(
        [aa_ref[0] + aa_ref[1] - ha_ref[...],
         ab_ref[0] + ab_ref[1] - hb_ref[...]], axis=1)
    h = jnp.maximum(agg * dis[:, None] + b_ref[...], 0.0)
    res = jnp.dot(h, w_ref[...],
                  preferred_element_type=jnp.float32) * dis[:, None]
    oa_ref[...] = res[:, :DH]
    ob_ref[...] = res[:, DH:]


def _fin_body(dp_ref, aa_ref, ab_ref, ha_ref, hb_ref, b_ref, o_ref):
    dis = _dis_from(dp_ref)
    agg = jnp.concatenate(
        [aa_ref[0] + aa_ref[1] - ha_ref[...],
         ab_ref[0] + ab_ref[1] - hb_ref[...]], axis=1)
    o_ref[...] = agg * dis[:, None] + b_ref[...]


_dp_spec = pl.BlockSpec((NC, BR, DEGW), lambda i: (0, i, 0))
_row_spec = pl.BlockSpec((BR, D), lambda i: (i, 0))
_half_spec = pl.BlockSpec((BR, DH), lambda i: (i, 0))
_acc_spec = pl.BlockSpec((NC, BR, DH), lambda i: (0, i, 0))
_w_spec = pl.BlockSpec((D, D), lambda i: (0, 0))
_b_spec = pl.BlockSpec((1, D), lambda i: (0, 0))
_half_sds = jax.ShapeDtypeStruct((N, DH), jnp.float32)
_full_sds = jax.ShapeDtypeStruct((N, D), jnp.float32)

_lin1_call = pl.pallas_call(
    _lin1_body, grid=(N // BR,),
    in_specs=[_dp_spec, _row_spec, _w_spec],
    out_specs=[_half_spec, _half_spec], out_shape=[_half_sds, _half_sds])

_lin2_call = pl.pallas_call(
    _lin2_body, grid=(N // BR,),
    in_specs=[_dp_spec, _acc_spec, _acc_spec, _half_spec, _half_spec,
              _b_spec, _w_spec],
    out_specs=[_half_spec, _half_spec], out_shape=[_half_sds, _half_sds])

_fin_call = pl.pallas_call(
    _fin_body, grid=(N // BR,),
    in_specs=[_dp_spec, _acc_spec, _acc_spec, _half_spec, _half_spec, _b_spec],
    out_specs=_row_spec, out_shape=_full_sds)


def kernel(x, edge_index, W1, b1, W2, b2):
    ei = edge_index.astype(jnp.int32)
    row = ei[0].reshape(NW, NCHUNK, K)
    col = ei[1].reshape(NW, NCHUNK, K)
    b1r = b1.reshape(1, D)
    b2r = b2.reshape(1, D)

    deg_parts = _deg_call(col)
    h1a, h1b = _lin1_call(deg_parts, x, W1)
    a1a = _agg_call(h1a, row, col)
    a1b = _agg_call(h1b, row, col)
    h2a, h2b = _lin2_call(deg_parts, a1a, a1b, h1a, h1b, b1r, W2)
    a2a = _agg_call(h2a, row, col)
    a2b = _agg_call(h2b, row, col)
    out = _fin_call(deg_parts, a2a, a2b, h2a, h2b, b2r)
    return out
